# baseline (speedup 1.0000x reference)
import functools

import jax
import jax.numpy as jnp
from jax import lax
from jax.experimental import pallas as pl
from jax.experimental.pallas import tpu as pltpu
from jax.experimental.pallas import tpu_sc as plsc

A = 128
E = 16
C = 256
M = 32
N = 10000
KC = A + E
EPS = 1e-5
SLOPE = 0.01

BN = 200
NB = N // BN
R = BN * M

NC = 2
NS = 16
NW = NC * NS
PER_W = (N * M) // NW
CH = 80
NCH = PER_W // CH


def _leaky(x):
    return jnp.where(x >= 0, x, SLOPE * x)


@functools.cache
def _make_sc_gather():
    mesh = plsc.VectorSubcoreMesh(core_axis_name="c", subcore_axis_name="s")

    @functools.partial(
        pl.kernel,
        mesh=mesh,
        out_type=jax.ShapeDtypeStruct((N * M, A), jnp.float32),
        scratch_types=[
            pltpu.VMEM((NCH, CH), jnp.int32),
            pltpu.VMEM((CH, A), jnp.float32),
            pltpu.SemaphoreType.DMA,
        ],
    )
    def _sc_gather(atom_hbm, idx_hbm, out_hbm, idx_v, rows_v, sem):
        wid = lax.axis_index("s") * NC + lax.axis_index("c")
        base = wid * PER_W
        pltpu.sync_copy(idx_hbm.at[wid], idx_v)

        def body(i, _):
            pltpu.async_copy(atom_hbm.at[idx_v.at[i]], rows_v, sem).wait()
            pltpu.sync_copy(rows_v, out_hbm.at[pl.ds(base + i * CH, CH)])
            return 0

        lax.fori_loop(0, NCH, body, 0)

    return _sc_gather


def _act_block(anbr_ref, nbr_ref, atom_ref, wst_ref, b_ref, wcat_ref):
    s_blk = (
        jnp.dot(atom_ref[...].astype(jnp.bfloat16), wst_ref[...],
                preferred_element_type=jnp.float32)
        + b_ref[...]
    )
    x = jnp.concatenate(
        [anbr_ref[...].astype(jnp.bfloat16), nbr_ref[...].astype(jnp.bfloat16)],
        axis=1)
    act = jnp.dot(x, wcat_ref[...], preferred_element_type=jnp.float32)
    act = act.reshape(BN, M, C) + s_blk[:, None, :]
    return act.reshape(R, C)


def _stats_body(anbr_ref, nbr_ref, atom_ref, wst_ref, b_ref, wcat_ref, stats_ref):
    i = pl.program_id(0)
    act = _act_block(anbr_ref, nbr_ref, atom_ref, wst_ref, b_ref, wcat_ref)
    s1 = jnp.sum(act, axis=0, keepdims=True)
    s2 = jnp.sum(act * act, axis=0, keepdims=True)
    st = jnp.concatenate([s1, s2], axis=0)

    @pl.when(i == 0)
    def _():
        stats_ref[...] = jnp.zeros_like(stats_ref)

    stats_ref[...] += st


def _gate_body(anbr_ref, nbr_ref, atom_ref, wst_ref, b_ref, wcat_ref,
               stats_ref, g1_ref, b1_ref, ns_ref, st2_ref):
    i = pl.program_id(0)
    act = _act_block(anbr_ref, nbr_ref, atom_ref, wst_ref, b_ref, wcat_ref)
    cnt = float(N * M)
    mean = stats_ref[0:1, :] / cnt
    var = stats_ref[1:2, :] / cnt - mean * mean
    scale = g1_ref[...] * lax.rsqrt(var + EPS)
    shift = b1_ref[...] - mean * scale
    y = act * scale + shift
    f = jax.nn.sigmoid(y[:, :A])
    co = _leaky(y[:, A:])
    ns = (f * co).reshape(BN, M, A).sum(axis=1)
    ns_ref[...] = ns
    s1 = jnp.sum(ns, axis=0, keepdims=True)
    s2 = jnp.sum(ns * ns, axis=0, keepdims=True)
    st = jnp.concatenate([s1, s2], axis=0)

    @pl.when(i == 0)
    def _():
        st2_ref[...] = jnp.zeros_like(st2_ref)

    st2_ref[...] += st


def _final_body(atom_ref, ns_ref, st2_ref, g2_ref, b2_ref, out_ref):
    cnt = float(N)
    mean = st2_ref[0:1, :] / cnt
    var = st2_ref[1:2, :] / cnt - mean * mean
    scale = g2_ref[...] * lax.rsqrt(var + EPS)
    shift = b2_ref[...] - mean * scale
    v = atom_ref[...] + ns_ref[...] * scale + shift
    out_ref[...] = _leaky(v)


def kernel(atom_in_fea, nbr_fea, nbr_fea_idx, W_fc, b_fc,
           bn1_gamma, bn1_beta, bn2_gamma, bn2_beta):
    idx = nbr_fea_idx.astype(jnp.int32).reshape(NW, NCH, CH)
    nbr_flat = nbr_fea.reshape(N * M, E)
    wst = W_fc[:, :A].T.astype(jnp.bfloat16)
    wcat = jnp.concatenate([W_fc[:, A:2 * A], W_fc[:, 2 * A:]],
                           axis=1).T.astype(jnp.bfloat16)
    b2d = b_fc.reshape(1, C)
    g1 = bn1_gamma.reshape(1, C)
    b1 = bn1_beta.reshape(1, C)
    g2 = bn2_gamma.reshape(1, A)
    b2 = bn2_beta.reshape(1, A)

    anbr = _make_sc_gather()(atom_in_fea, idx)

    edge_specs = [
        pl.BlockSpec((R, A), lambda i: (i, 0)),
        pl.BlockSpec((R, E), lambda i: (i, 0)),
        pl.BlockSpec((BN, A), lambda i: (i, 0)),
        pl.BlockSpec((A, C), lambda i: (0, 0)),
        pl.BlockSpec((1, C), lambda i: (0, 0)),
        pl.BlockSpec((KC, C), lambda i: (0, 0)),
    ]

    stats = pl.pallas_call(
        _stats_body,
        grid=(NB,),
        in_specs=edge_specs,
        out_specs=pl.BlockSpec((2, C), lambda i: (0, 0)),
        out_shape=jax.ShapeDtypeStruct((2, C), jnp.float32),
    )(anbr, nbr_flat, atom_in_fea, wst, b2d, wcat)

    ns, st2 = pl.pallas_call(
        _gate_body,
        grid=(NB,),
        in_specs=edge_specs + [
            pl.BlockSpec((2, C), lambda i: (0, 0)),
            pl.BlockSpec((1, C), lambda i: (0, 0)),
            pl.BlockSpec((1, C), lambda i: (0, 0)),
        ],
        out_specs=[
            pl.BlockSpec((BN, A), lambda i: (i, 0)),
            pl.BlockSpec((2, A), lambda i: (0, 0)),
        ],
        out_shape=[
            jax.ShapeDtypeStruct((N, A), jnp.float32),
            jax.ShapeDtypeStruct((2, A), jnp.float32),
        ],
    )(anbr, nbr_flat, atom_in_fea, wst, b2d, wcat, stats, g1, b1)

    out = pl.pallas_call(
        _final_body,
        in_specs=[
            pl.BlockSpec((N, A), lambda: (0, 0)),
            pl.BlockSpec((N, A), lambda: (0, 0)),
            pl.BlockSpec((2, A), lambda: (0, 0)),
            pl.BlockSpec((1, A), lambda: (0, 0)),
            pl.BlockSpec((1, A), lambda: (0, 0)),
        ],
        out_specs=pl.BlockSpec((N, A), lambda: (0, 0)),
        out_shape=jax.ShapeDtypeStruct((N, A), jnp.float32),
    )(atom_in_fea, ns, st2, g2, b2)
    return out


# speedup vs baseline: 1.1660x; 1.1660x over previous
import functools

import jax
import jax.numpy as jnp
from jax import lax
from jax.experimental import pallas as pl
from jax.experimental.pallas import tpu as pltpu
from jax.experimental.pallas import tpu_sc as plsc

A = 128
E = 16
C = 256
M = 32
N = 10000
KC = A + E
EPS = 1e-5
SLOPE = 0.01

BN = 200
NB = N // BN
R = BN * M

NC = 2
NS = 16
NW = NC * NS
PER_W = (N * M) // NW
CH = 80
NCH = PER_W // CH


def _leaky(x):
    return jnp.where(x >= 0, x, SLOPE * x)


@functools.cache
def _make_sc_gather():
    mesh = plsc.VectorSubcoreMesh(core_axis_name="c", subcore_axis_name="s")

    @functools.partial(
        pl.kernel,
        mesh=mesh,
        out_type=jax.ShapeDtypeStruct((N * M, A), jnp.float32),
        scratch_types=[
            pltpu.VMEM((PER_W,), jnp.int32),
            pltpu.VMEM((2, CH, A), jnp.float32),
            pltpu.SemaphoreType.DMA,
            pltpu.SemaphoreType.DMA,
        ],
    )
    def _sc_gather(atom_hbm, idx_hbm, out_hbm, idx_v, rows_v, sem0, sem1):
        wid = lax.axis_index("s") * NC + lax.axis_index("c")
        base = wid * PER_W
        pltpu.sync_copy(idx_hbm.at[pl.ds(base, PER_W)], idx_v)
        sems = (sem0, sem1)

        def chunk_idx(i):
            return idx_v.at[pl.ds(i * CH, CH)]

        def start(c, slot):
            pltpu.async_copy(atom_hbm.at[chunk_idx(c)], rows_v.at[slot],
                             sems[slot])

        def drain_store(c, slot):
            pltpu.make_async_copy(atom_hbm.at[chunk_idx(c)], rows_v.at[slot],
                                  sems[slot]).wait()
            pltpu.sync_copy(rows_v.at[slot],
                            out_hbm.at[pl.ds(base + c * CH, CH)])

        start(0, 0)

        def body(j, _):
            start(2 * j + 1, 1)
            drain_store(2 * j, 0)
            start(2 * j + 2, 0)
            drain_store(2 * j + 1, 1)
            return 0

        lax.fori_loop(0, (NCH - 1) // 2, body, 0)
        drain_store(NCH - 1, 0)

    return _sc_gather


def _act_block(anbr_ref, nbr_ref, atom_ref, wst_ref, b_ref, wcat_ref):
    s_blk = (
        jnp.dot(atom_ref[...].astype(jnp.bfloat16), wst_ref[...],
                preferred_element_type=jnp.float32)
        + b_ref[...]
    )
    x = jnp.concatenate(
        [anbr_ref[...].astype(jnp.bfloat16), nbr_ref[...].astype(jnp.bfloat16)],
        axis=1)
    act = jnp.dot(x, wcat_ref[...], preferred_element_type=jnp.float32)
    act = act.reshape(BN, M, C) + s_blk[:, None, :]
    return act.reshape(R, C)


def _stats_body(anbr_ref, nbr_ref, atom_ref, wst_ref, b_ref, wcat_ref, stats_ref):
    i = pl.program_id(0)
    act = _act_block(anbr_ref, nbr_ref, atom_ref, wst_ref, b_ref, wcat_ref)
    s1 = jnp.sum(act, axis=0, keepdims=True)
    s2 = jnp.sum(act * act, axis=0, keepdims=True)
    st = jnp.concatenate([s1, s2], axis=0)

    @pl.when(i == 0)
    def _():
        stats_ref[...] = jnp.zeros_like(stats_ref)

    stats_ref[...] += st


def _gate_body(anbr_ref, nbr_ref, atom_ref, wst_ref, b_ref, wcat_ref,
               stats_ref, g1_ref, b1_ref, ns_ref, st2_ref):
    i = pl.program_id(0)
    act = _act_block(anbr_ref, nbr_ref, atom_ref, wst_ref, b_ref, wcat_ref)
    cnt = float(N * M)
    mean = stats_ref[0:1, :] / cnt
    var = stats_ref[1:2, :] / cnt - mean * mean
    scale = g1_ref[...] * lax.rsqrt(var + EPS)
    shift = b1_ref[...] - mean * scale
    y = act * scale + shift
    f = jax.nn.sigmoid(y[:, :A])
    co = _leaky(y[:, A:])
    ns = (f * co).reshape(BN, M, A).sum(axis=1)
    ns_ref[...] = ns
    s1 = jnp.sum(ns, axis=0, keepdims=True)
    s2 = jnp.sum(ns * ns, axis=0, keepdims=True)
    st = jnp.concatenate([s1, s2], axis=0)

    @pl.when(i == 0)
    def _():
        st2_ref[...] = jnp.zeros_like(st2_ref)

    st2_ref[...] += st


def _final_body(atom_ref, ns_ref, st2_ref, g2_ref, b2_ref, out_ref):
    cnt = float(N)
    mean = st2_ref[0:1, :] / cnt
    var = st2_ref[1:2, :] / cnt - mean * mean
    scale = g2_ref[...] * lax.rsqrt(var + EPS)
    shift = b2_ref[...] - mean * scale
    v = atom_ref[...] + ns_ref[...] * scale + shift
    out_ref[...] = _leaky(v)


def kernel(atom_in_fea, nbr_fea, nbr_fea_idx, W_fc, b_fc,
           bn1_gamma, bn1_beta, bn2_gamma, bn2_beta):
    idx = nbr_fea_idx.astype(jnp.int32).reshape(N * M)
    nbr_flat = nbr_fea.reshape(N * M, E)
    wst = W_fc[:, :A].T.astype(jnp.bfloat16)
    wcat = jnp.concatenate([W_fc[:, A:2 * A], W_fc[:, 2 * A:]],
                           axis=1).T.astype(jnp.bfloat16)
    b2d = b_fc.reshape(1, C)
    g1 = bn1_gamma.reshape(1, C)
    b1 = bn1_beta.reshape(1, C)
    g2 = bn2_gamma.reshape(1, A)
    b2 = bn2_beta.reshape(1, A)

    anbr = _make_sc_gather()(atom_in_fea, idx)

    edge_specs = [
        pl.BlockSpec((R, A), lambda i: (i, 0)),
        pl.BlockSpec((R, E), lambda i: (i, 0)),
        pl.BlockSpec((BN, A), lambda i: (i, 0)),
        pl.BlockSpec((A, C), lambda i: (0, 0)),
        pl.BlockSpec((1, C), lambda i: (0, 0)),
        pl.BlockSpec((KC, C), lambda i: (0, 0)),
    ]

    stats = pl.pallas_call(
        _stats_body,
        grid=(NB,),
        in_specs=edge_specs,
        out_specs=pl.BlockSpec((2, C), lambda i: (0, 0)),
        out_shape=jax.ShapeDtypeStruct((2, C), jnp.float32),
    )(anbr, nbr_flat, atom_in_fea, wst, b2d, wcat)

    ns, st2 = pl.pallas_call(
        _gate_body,
        grid=(NB,),
        in_specs=edge_specs + [
            pl.BlockSpec((2, C), lambda i: (0, 0)),
            pl.BlockSpec((1, C), lambda i: (0, 0)),
            pl.BlockSpec((1, C), lambda i: (0, 0)),
        ],
        out_specs=[
            pl.BlockSpec((BN, A), lambda i: (i, 0)),
            pl.BlockSpec((2, A), lambda i: (0, 0)),
        ],
        out_shape=[
            jax.ShapeDtypeStruct((N, A), jnp.float32),
            jax.ShapeDtypeStruct((2, A), jnp.float32),
        ],
    )(anbr, nbr_flat, atom_in_fea, wst, b2d, wcat, stats, g1, b1)

    out = pl.pallas_call(
        _final_body,
        in_specs=[
            pl.BlockSpec((N, A), lambda: (0, 0)),
            pl.BlockSpec((N, A), lambda: (0, 0)),
            pl.BlockSpec((2, A), lambda: (0, 0)),
            pl.BlockSpec((1, A), lambda: (0, 0)),
            pl.BlockSpec((1, A), lambda: (0, 0)),
        ],
        out_specs=pl.BlockSpec((N, A), lambda: (0, 0)),
        out_shape=jax.ShapeDtypeStruct((N, A), jnp.float32),
    )(atom_in_fea, ns, st2, g2, b2)
    return out


# speedup vs baseline: 1.2168x; 1.0435x over previous
import functools

import jax
import jax.numpy as jnp
from jax import lax
from jax.experimental import pallas as pl
from jax.experimental.pallas import tpu as pltpu
from jax.experimental.pallas import tpu_sc as plsc

A = 128
E = 16
C = 256
M = 32
N = 10000
KC = A + E
EPS = 1e-5
SLOPE = 0.01

BN = 200
NB = N // BN
R = BN * M

NC = 2
NS = 16
NW = NC * NS
H = 2
ROWS_H = (N * M) // H
PER_W = ROWS_H // NW
CH = 40
NCH = PER_W // CH
NBH = NB // H


def _leaky(x):
    return jnp.where(x >= 0, x, SLOPE * x)


@functools.cache
def _make_sc_gather():
    mesh = plsc.VectorSubcoreMesh(core_axis_name="c", subcore_axis_name="s")

    @functools.partial(
        pl.kernel,
        mesh=mesh,
        out_type=jax.ShapeDtypeStruct((ROWS_H, A), jnp.float32),
        scratch_types=[
            pltpu.VMEM((PER_W,), jnp.int32),
            pltpu.VMEM((2, CH, A), jnp.float32),
            pltpu.SemaphoreType.DMA,
            pltpu.SemaphoreType.DMA,
        ],
    )
    def _sc_gather(atom_hbm, idx_hbm, out_hbm, idx_v, rows_v, sem0, sem1):
        wid = lax.axis_index("s") * NC + lax.axis_index("c")
        base = wid * PER_W
        pltpu.sync_copy(idx_hbm.at[pl.ds(base, PER_W)], idx_v)
        sems = (sem0, sem1)

        def chunk_idx(i):
            return idx_v.at[pl.ds(i * CH, CH)]

        def start(c, slot):
            pltpu.async_copy(atom_hbm.at[chunk_idx(c)], rows_v.at[slot],
                             sems[slot])

        def drain_store(c, slot):
            pltpu.make_async_copy(atom_hbm.at[chunk_idx(c)], rows_v.at[slot],
                                  sems[slot]).wait()
            pltpu.sync_copy(rows_v.at[slot],
                            out_hbm.at[pl.ds(base + c * CH, CH)])

        start(0, 0)

        def body(j, _):
            start(2 * j + 1, 1)
            drain_store(2 * j, 0)
            start(2 * j + 2, 0)
            drain_store(2 * j + 1, 1)
            return 0

        lax.fori_loop(0, (NCH - 1) // 2, body, 0)
        drain_store(NCH - 1, 0)

    return _sc_gather


def _act_block(anbr_ref, nbr_ref, atom_ref, wst_ref, b_ref, wcat_ref):
    s_blk = (
        jnp.dot(atom_ref[...].astype(jnp.bfloat16), wst_ref[...],
                preferred_element_type=jnp.float32)
        + b_ref[...]
    )
    x = jnp.concatenate([anbr_ref[...].astype(jnp.bfloat16), nbr_ref[...]], axis=1)
    act = jnp.dot(x, wcat_ref[...], preferred_element_type=jnp.float32)
    act = act.reshape(BN, M, C) + s_blk[:, None, :]
    return act.reshape(R, C)


def _stats_body(anbr_ref, nbr_ref, atom_ref, wst_ref, b_ref, wcat_ref, stats_ref):
    i = pl.program_id(0)
    act = _act_block(anbr_ref, nbr_ref, atom_ref, wst_ref, b_ref, wcat_ref)
    s1 = jnp.sum(act, axis=0, keepdims=True)
    s2 = jnp.sum(act * act, axis=0, keepdims=True)
    st = jnp.concatenate([s1, s2], axis=0)

    @pl.when(i == 0)
    def _():
        stats_ref[...] = jnp.zeros_like(stats_ref)

    stats_ref[...] += st


def _gate_body(anbr_ref, nbr_ref, atom_ref, wst_ref, b_ref, wcat_ref,
               stats_ref, g1_ref, b1_ref, ns_ref, st2_ref):
    i = pl.program_id(0)
    act = _act_block(anbr_ref, nbr_ref, atom_ref, wst_ref, b_ref, wcat_ref)
    cnt = float(N * M)
    mean = stats_ref[0:1, :] / cnt
    var = stats_ref[1:2, :] / cnt - mean * mean
    scale = g1_ref[...] * lax.rsqrt(var + EPS)
    shift = b1_ref[...] - mean * scale
    y = act * scale + shift
    f = jax.nn.sigmoid(y[:, :A])
    co = _leaky(y[:, A:])
    ns = (f * co).reshape(BN, M, A).sum(axis=1)
    ns_ref[...] = ns
    s1 = jnp.sum(ns, axis=0, keepdims=True)
    s2 = jnp.sum(ns * ns, axis=0, keepdims=True)
    st = jnp.concatenate([s1, s2], axis=0)

    @pl.when(i == 0)
    def _():
        st2_ref[...] = jnp.zeros_like(st2_ref)

    st2_ref[...] += st


def _final_body(atom_ref, ns_ref, st2_ref, g2_ref, b2_ref, out_ref):
    cnt = float(N)
    mean = st2_ref[0:1, :] / cnt
    var = st2_ref[1:2, :] / cnt - mean * mean
    scale = g2_ref[...] * lax.rsqrt(var + EPS)
    shift = b2_ref[...] - mean * scale
    v = atom_ref[...] + ns_ref[...] * scale + shift
    out_ref[...] = _leaky(v)


def kernel(atom_in_fea, nbr_fea, nbr_fea_idx, W_fc, b_fc,
           bn1_gamma, bn1_beta, bn2_gamma, bn2_beta):
    idx = nbr_fea_idx.astype(jnp.int32).reshape(N * M)
    nbr_flat = nbr_fea.reshape(N * M, E).astype(jnp.bfloat16)
    wst = W_fc[:, :A].T.astype(jnp.bfloat16)
    wcat = jnp.concatenate([W_fc[:, A:2 * A], W_fc[:, 2 * A:]],
                           axis=1).T.astype(jnp.bfloat16)
    b2d = b_fc.reshape(1, C)
    g1 = bn1_gamma.reshape(1, C)
    b1 = bn1_beta.reshape(1, C)
    g2 = bn2_gamma.reshape(1, A)
    b2 = bn2_beta.reshape(1, A)

    gather = _make_sc_gather()
    anbr_h = [gather(atom_in_fea, idx[h * ROWS_H:(h + 1) * ROWS_H])
              for h in range(H)]

    def edge_specs(h):
        return [
            pl.BlockSpec((R, A), lambda i: (i, 0)),
            pl.BlockSpec((R, E), lambda i, h=h: (i + h * NBH, 0)),
            pl.BlockSpec((BN, A), lambda i, h=h: (i + h * NBH, 0)),
            pl.BlockSpec((A, C), lambda i: (0, 0)),
            pl.BlockSpec((1, C), lambda i: (0, 0)),
            pl.BlockSpec((KC, C), lambda i: (0, 0)),
        ]

    stats_h = [
        pl.pallas_call(
            _stats_body,
            grid=(NBH,),
            in_specs=edge_specs(h),
            out_specs=pl.BlockSpec((2, C), lambda i: (0, 0)),
            out_shape=jax.ShapeDtypeStruct((2, C), jnp.float32),
        )(anbr_h[h], nbr_flat, atom_in_fea, wst, b2d, wcat)
        for h in range(H)
    ]
    stats = stats_h[0] + stats_h[1]

    ns_h, st2_h = [], []
    for h in range(H):
        ns, st2 = pl.pallas_call(
            _gate_body,
            grid=(NBH,),
            in_specs=edge_specs(h) + [
                pl.BlockSpec((2, C), lambda i: (0, 0)),
                pl.BlockSpec((1, C), lambda i: (0, 0)),
                pl.BlockSpec((1, C), lambda i: (0, 0)),
            ],
            out_specs=[
                pl.BlockSpec((BN, A), lambda i: (i, 0)),
                pl.BlockSpec((2, A), lambda i: (0, 0)),
            ],
            out_shape=[
                jax.ShapeDtypeStruct((N // H, A), jnp.float32),
                jax.ShapeDtypeStruct((2, A), jnp.float32),
            ],
        )(anbr_h[h], nbr_flat, atom_in_fea, wst, b2d, wcat, stats, g1, b1)
        ns_h.append(ns)
        st2_h.append(st2)
    ns = jnp.concatenate(ns_h, axis=0)
    st2 = st2_h[0] + st2_h[1]

    out = pl.pallas_call(
        _final_body,
        in_specs=[
            pl.BlockSpec((N, A), lambda: (0, 0)),
            pl.BlockSpec((N, A), lambda: (0, 0)),
            pl.BlockSpec((2, A), lambda: (0, 0)),
            pl.BlockSpec((1, A), lambda: (0, 0)),
            pl.BlockSpec((1, A), lambda: (0, 0)),
        ],
        out_specs=pl.BlockSpec((N, A), lambda: (0, 0)),
        out_shape=jax.ShapeDtypeStruct((N, A), jnp.float32),
    )(atom_in_fea, ns, st2, g2, b2)
    return out
